# baseline (speedup 1.0000x reference)
import functools
import math

import jax
import jax.numpy as jnp
from jax import lax
from jax.experimental import pallas as pl
from jax.experimental.pallas import tpu as pltpu
from jax.experimental.pallas import tpu_sc as plsc

EMBED_DIM = 64
PAD_DIM = 128
SCALE = math.sqrt(EMBED_DIM)

NUM_CORES = 2
NUM_SUBCORES = 16
NUM_WORKERS = NUM_CORES * NUM_SUBCORES
LANES = 16

CHUNK = 200
ROWS_PER_ITER = 4


def _make_kernel(batch: int):
    assert batch % NUM_WORKERS == 0
    rows_per_worker = batch // NUM_WORKERS
    assert rows_per_worker % (2 * CHUNK) == 0
    n_chunks = rows_per_worker // CHUNK

    mesh = plsc.VectorSubcoreMesh(
        core_axis_name="c", subcore_axis_name="s"
    )

    @functools.partial(
        pl.kernel,
        mesh=mesh,
        out_type=jax.ShapeDtypeStruct((batch, EMBED_DIM), jnp.float32),
        scratch_types=[
            pltpu.VMEM((CHUNK,), jnp.int32),
            pltpu.VMEM((CHUNK,), jnp.int32),
            pltpu.VMEM((CHUNK, PAD_DIM), jnp.float32),
            pltpu.VMEM((CHUNK, PAD_DIM), jnp.float32),
            pltpu.VMEM((CHUNK, EMBED_DIM), jnp.float32),
            pltpu.VMEM((CHUNK, EMBED_DIM), jnp.float32),
            pltpu.SemaphoreType.DMA,
            pltpu.SemaphoreType.DMA,
            pltpu.SemaphoreType.DMA,
            pltpu.SemaphoreType.DMA,
        ],
    )
    def emb_kernel(x_hbm, table_hbm, out_hbm, idx0, idx1, rw0, rw1,
                   rn0, rn1, sg0, sg1, ss0, ss1):
        wid = lax.axis_index("s") * NUM_CORES + lax.axis_index("c")
        base = wid * rows_per_worker

        idx_v = (idx0, idx1)
        wide_v = (rw0, rw1)
        narrow_v = (rn0, rn1)
        sg = (sg0, sg1)
        ss = (ss0, ss1)

        def start_gather(c, bf):
            off = base + c * CHUNK
            pltpu.sync_copy(x_hbm.at[pl.ds(off, CHUNK)], idx_v[bf])
            pltpu.async_copy(table_hbm.at[idx_v[bf]], wide_v[bf], sg[bf])

        def wait_gather(bf):
            pltpu.make_async_copy(
                table_hbm.at[idx_v[bf]], wide_v[bf], sg[bf]
            ).wait()

        def start_store(c, bf):
            off = base + c * CHUNK
            pltpu.async_copy(
                narrow_v[bf], out_hbm.at[pl.ds(off, CHUNK)], ss[bf]
            )

        def wait_store(bf):
            pltpu.make_async_copy(
                narrow_v[bf], out_hbm.at[pl.ds(base, CHUNK)], ss[bf]
            ).wait()

        def scale_rows(bf):
            wv = wide_v[bf]
            nv = narrow_v[bf]

            def body(r0, carry):
                r = r0 * ROWS_PER_ITER
                for dr in range(ROWS_PER_ITER):
                    for q in range(EMBED_DIM // LANES):
                        sl = wv[r + dr, pl.ds(q * LANES, LANES)]
                        nv[r + dr, pl.ds(q * LANES, LANES)] = sl * SCALE
                return carry

            lax.fori_loop(0, CHUNK // ROWS_PER_ITER, body, 0)

        start_gather(0, 0)

        def pair_body(p, carry):
            for b2 in range(2):
                c = p * 2 + b2
                bf = b2
                nb = 1 - b2

                @pl.when(c + 1 < n_chunks)
                def _():
                    start_gather(c + 1, nb)

                wait_gather(bf)

                @pl.when(c >= 2)
                def _():
                    wait_store(bf)

                scale_rows(bf)
                start_store(c, bf)
            return carry

        lax.fori_loop(0, n_chunks // 2, pair_body, 0)
        wait_store(0)
        wait_store(1)

    return emb_kernel


def kernel(x, table):
    b, h = x.shape
    batch = b * h
    xf = x.reshape(batch)
    tpad = jnp.concatenate([table, table], axis=1)
    out = _make_kernel(batch)(xf, tpad)
    return out.reshape(b, h, EMBED_DIM)


# speedup vs baseline: 1.1519x; 1.1519x over previous
import functools
import math

import jax
import jax.numpy as jnp
from jax import lax
from jax.experimental import pallas as pl
from jax.experimental.pallas import tpu as pltpu
from jax.experimental.pallas import tpu_sc as plsc

EMBED_DIM = 64
PAD_DIM = 128
SCALE = math.sqrt(EMBED_DIM)

NUM_CORES = 2
NUM_SUBCORES = 16
NUM_WORKERS = NUM_CORES * NUM_SUBCORES
LANES = 16

CHUNK = 200
ROWS_PER_ITER = 4


def _make_kernel(batch: int):
    assert batch % NUM_WORKERS == 0
    rows_per_worker = batch // NUM_WORKERS
    assert rows_per_worker % (2 * CHUNK) == 0
    n_chunks = rows_per_worker // CHUNK

    mesh = plsc.VectorSubcoreMesh(
        core_axis_name="c", subcore_axis_name="s"
    )

    @functools.partial(
        pl.kernel,
        mesh=mesh,
        out_type=jax.ShapeDtypeStruct((batch, EMBED_DIM), jnp.float32),
        scratch_types=[
            pltpu.VMEM((CHUNK,), jnp.int32),
            pltpu.VMEM((CHUNK,), jnp.int32),
            pltpu.VMEM((CHUNK, PAD_DIM), jnp.float32),
            pltpu.VMEM((CHUNK, PAD_DIM), jnp.float32),
            pltpu.VMEM((CHUNK, EMBED_DIM), jnp.float32),
            pltpu.VMEM((CHUNK, EMBED_DIM), jnp.float32),
            pltpu.SemaphoreType.DMA,
            pltpu.SemaphoreType.DMA,
            pltpu.SemaphoreType.DMA,
            pltpu.SemaphoreType.DMA,
        ],
    )
    def emb_kernel(x_hbm, table_hbm, out_hbm, idx0, idx1, rw0, rw1,
                   rn0, rn1, sg0, sg1, ss0, ss1):
        wid = lax.axis_index("s") * NUM_CORES + lax.axis_index("c")
        base = wid * rows_per_worker

        idx_v = (idx0, idx1)
        wide_v = (rw0, rw1)
        narrow_v = (rn0, rn1)
        sg = (sg0, sg1)
        ss = (ss0, ss1)

        def start_gather(c, bf):
            off = base + c * CHUNK
            pltpu.sync_copy(x_hbm.at[pl.ds(off, CHUNK)], idx_v[bf])
            pltpu.async_copy(table_hbm.at[idx_v[bf]], wide_v[bf], sg[bf])

        def wait_gather(bf):
            pltpu.make_async_copy(
                table_hbm.at[idx_v[bf]], wide_v[bf], sg[bf]
            ).wait()

        def start_store(c, bf):
            off = base + c * CHUNK
            pltpu.async_copy(
                narrow_v[bf], out_hbm.at[pl.ds(off, CHUNK)], ss[bf]
            )

        def wait_store(bf):
            pltpu.make_async_copy(
                narrow_v[bf], out_hbm.at[pl.ds(base, CHUNK)], ss[bf]
            ).wait()

        def scale_rows(bf):
            wv = wide_v[bf]
            nv = narrow_v[bf]

            def body(r0, carry):
                r = r0 * ROWS_PER_ITER
                for dr in range(ROWS_PER_ITER):
                    for q in range(EMBED_DIM // LANES):
                        sl = wv[r + dr, pl.ds(q * LANES, LANES)]
                        nv[r + dr, pl.ds(q * LANES, LANES)] = sl * SCALE
                return carry

            lax.fori_loop(0, CHUNK // ROWS_PER_ITER, body, 0)

        start_gather(0, 0)

        def pair_body(p, carry):
            for b2 in range(2):
                c = p * 2 + b2
                bf = b2
                nb = 1 - b2

                @pl.when(c + 1 < n_chunks)
                def _():
                    start_gather(c + 1, nb)

                wait_gather(bf)

                @pl.when(c >= 2)
                def _():
                    wait_store(bf)

                scale_rows(bf)
                start_store(c, bf)
            return carry

        lax.fori_loop(0, n_chunks // 2, pair_body, 0)
        wait_store(0)
        wait_store(1)

    return emb_kernel


def kernel(x, table):
    b, h = x.shape
    batch = b * h
    xf = x.reshape(batch)
    tpad = jnp.pad(table, ((0, 0), (0, PAD_DIM - EMBED_DIM)))
    out = _make_kernel(batch)(xf, tpad)
    return out.reshape(b, h, EMBED_DIM)
